# baseline (speedup 1.0000x reference)
import jax
import jax.numpy as jnp
from jax import lax
from jax.experimental import pallas as pl
from jax.experimental.pallas import tpu as pltpu
from jax.experimental.pallas import tpu_sc as plsc

_NC = 2
_NS = 16
_NW = _NC * _NS
_CHUNK = 32
_G = 8
_NBUF = 4


def _sc_segsum(n_pad, d, k):
  rps = n_pad // _NS

  def body(x_hbm, src_hbm, dst_hbm, agg_hbm,
           idx_s, idx_d, r0, r1, r2, r3, agg_sh, s0, s1, s2, s3):
    cid = lax.axis_index("c")
    sid = lax.axis_index("s")
    w = cid * _NS + sid
    row_bufs = (r0, r1, r2, r3)
    sems = (s0, s1, s2, s3)

    def zrow(i, _):
      for j in range(d // 16):
        r0[i, pl.ds(j * 16, 16)] = jnp.zeros((16,), jnp.float32)
      return 0
    lax.fori_loop(0, _CHUNK, zrow, 0)
    for t in range(rps // _CHUNK):
      pltpu.sync_copy(r0, agg_sh.at[pl.ds(sid * rps + t * _CHUNK, _CHUNK)])
    plsc.subcore_barrier()

    def stage(g):
      gp = g % 2
      pltpu.sync_copy(src_hbm.at[w, pl.ds(g * _G, _G)], idx_s.at[gp])
      pltpu.sync_copy(dst_hbm.at[w, pl.ds(g * _G, _G)], idx_d.at[gp])

    def start_gather(c, b):
      isrc = idx_s.at[(c // _G) % 2, c % _G]
      pltpu.async_copy(x_hbm.at[isrc], row_bufs[b], sems[b])

    def finish(c, b):
      isrc = idx_s.at[(c // _G) % 2, c % _G]
      pltpu.make_async_copy(x_hbm.at[isrc], row_bufs[b], sems[b]).wait()
      pltpu.sync_copy(row_bufs[b], agg_sh.at[idx_d.at[(c // _G) % 2, c % _G]],
                      add=True)

    stage(0)
    for b in range(_NBUF):
      start_gather(b, b)

    nq = k // _NBUF
    def quad(q, _):
      c0 = _NBUF * q
      @pl.when(jnp.logical_and(q % 2 == 1, q < nq - 1))
      def _():
        stage((c0 + _NBUF) // _G)
      for u in range(_NBUF):
        finish(c0 + u, u)
        @pl.when(q < nq - 1)
        def _():
          start_gather(c0 + u + _NBUF, u)
      return 0
    lax.fori_loop(0, nq, quad, 0)
    plsc.subcore_barrier()

    for t in range(rps // _CHUNK):
      sl = pl.ds(sid * rps + t * _CHUNK, _CHUNK)
      pltpu.sync_copy(agg_sh.at[sl], r0)
      pltpu.sync_copy(r0, agg_hbm.at[cid, sl])

  mesh = plsc.VectorSubcoreMesh(
      core_axis_name="c", subcore_axis_name="s",
      num_cores=_NC, num_subcores=_NS)
  return pl.kernel(
      body,
      out_type=jax.ShapeDtypeStruct((_NC, n_pad, d), jnp.float32),
      mesh=mesh,
      scratch_types=[
          pltpu.VMEM((2, _G, _CHUNK), jnp.int32),
          pltpu.VMEM((2, _G, _CHUNK), jnp.int32),
          pltpu.VMEM((_CHUNK, d), jnp.float32),
          pltpu.VMEM((_CHUNK, d), jnp.float32),
          pltpu.VMEM((_CHUNK, d), jnp.float32),
          pltpu.VMEM((_CHUNK, d), jnp.float32),
          pltpu.VMEM_SHARED((n_pad, d), jnp.float32),
          pltpu.SemaphoreType.DMA,
          pltpu.SemaphoreType.DMA,
          pltpu.SemaphoreType.DMA,
          pltpu.SemaphoreType.DMA,
      ],
  )


def _sc_count(n_pad, d, k):
  rps = n_pad // _NS
  ng = k // _G

  def body(dst_hbm, cnt_hbm, idx_d, ones, cnt_sh):
    cid = lax.axis_index("c")
    sid = lax.axis_index("s")
    w = cid * _NS + sid

    def zrow(i, _):
      for j in range(d // 16):
        ones[i, pl.ds(j * 16, 16)] = jnp.zeros((16,), jnp.float32)
      return 0
    lax.fori_loop(0, _CHUNK, zrow, 0)
    for t in range(rps // _CHUNK):
      pltpu.sync_copy(ones, cnt_sh.at[pl.ds(sid * rps + t * _CHUNK, _CHUNK)])

    def orow(i, _):
      for j in range(d // 16):
        ones[i, pl.ds(j * 16, 16)] = jnp.ones((16,), jnp.float32)
      return 0
    lax.fori_loop(0, _CHUNK, orow, 0)
    plsc.subcore_barrier()

    def group(g, _):
      pltpu.sync_copy(dst_hbm.at[w, pl.ds(g * _G, _G)], idx_d)
      for j in range(_G):
        pltpu.sync_copy(ones, cnt_sh.at[idx_d.at[j]], add=True)
      return 0
    lax.fori_loop(0, ng, group, 0)
    plsc.subcore_barrier()

    for t in range(rps // _CHUNK):
      sl = pl.ds(sid * rps + t * _CHUNK, _CHUNK)
      pltpu.sync_copy(cnt_sh.at[sl], ones)
      pltpu.sync_copy(ones, cnt_hbm.at[cid, sl])

  mesh = plsc.VectorSubcoreMesh(
      core_axis_name="c", subcore_axis_name="s",
      num_cores=_NC, num_subcores=_NS)
  return pl.kernel(
      body,
      out_type=jax.ShapeDtypeStruct((_NC, n_pad, d), jnp.float32),
      mesh=mesh,
      scratch_types=[
          pltpu.VMEM((_G, _CHUNK), jnp.int32),
          pltpu.VMEM((_CHUNK, d), jnp.float32),
          pltpu.VMEM_SHARED((n_pad, d), jnp.float32),
      ],
  )


def _tc_layer1(agg_ref, cnt_ref, x_ref, wl_ref, wr_ref, b_ref, o_ref):
  agg = agg_ref[0] + agg_ref[1]
  cnt = cnt_ref[0, :, 0:1] + cnt_ref[1, :, 0:1]
  mean = agg / jnp.maximum(cnt, 1.0)
  acc = jnp.dot(mean, wl_ref[...], preferred_element_type=jnp.float32)
  acc += jnp.dot(x_ref[...], wr_ref[...], preferred_element_type=jnp.float32)
  o_ref[...] = jnp.maximum(acc + b_ref[...], 0.0)


def _tc_layer2(agg_ref, cnt_ref, h_ref, wl_ref, wr_ref, b_ref, wh_ref, bh_ref,
               o_ref):
  agg = agg_ref[0] + agg_ref[1]
  cnt = cnt_ref[0, :, 0:1] + cnt_ref[1, :, 0:1]
  mean = agg / jnp.maximum(cnt, 1.0)
  acc = jnp.dot(mean, wl_ref[...], preferred_element_type=jnp.float32)
  acc += jnp.dot(h_ref[...], wr_ref[...], preferred_element_type=jnp.float32)
  h2 = jnp.maximum(acc + b_ref[...], 0.0)
  o_ref[...] = jnp.dot(h2, wh_ref[...], preferred_element_type=jnp.float32) \
      + bh_ref[...]


def kernel(x, edge_index, W1l, b1, W1r, W2l, b2, W2r, Wh, bh):
  n, d = x.shape
  e = edge_index.shape[1]

  k = -(-e // (_NW * _CHUNK * _G)) * _G
  e_pad = _NW * _CHUNK * k
  n_pad = -(-n // (_NS * _CHUNK)) * (_NS * _CHUNK)

  src = jnp.pad(edge_index[0], (0, e_pad - e)).reshape(_NW, k, _CHUNK)
  dst = jnp.pad(edge_index[1], (0, e_pad - e),
                constant_values=n).reshape(_NW, k, _CHUNK)
  x_pad = jnp.pad(x, ((0, n_pad - n), (0, 0)))

  cnt = _sc_count(n_pad, d, k)(dst)
  agg1 = _sc_segsum(n_pad, d, k)(x, src, dst)

  blk = 512
  grid = (n_pad // blk,)
  h = pl.pallas_call(
      _tc_layer1,
      grid=grid,
      in_specs=[
          pl.BlockSpec((_NC, blk, d), lambda i: (0, i, 0)),
          pl.BlockSpec((_NC, blk, d), lambda i: (0, i, 0)),
          pl.BlockSpec((blk, d), lambda i: (i, 0)),
          pl.BlockSpec((d, d), lambda i: (0, 0)),
          pl.BlockSpec((d, d), lambda i: (0, 0)),
          pl.BlockSpec((1, d), lambda i: (0, 0)),
      ],
      out_specs=pl.BlockSpec((blk, d), lambda i: (i, 0)),
      out_shape=jax.ShapeDtypeStruct((n_pad, d), jnp.float32),
  )(agg1, cnt, x_pad, W1l, W1r, b1.reshape(1, d))

  agg2 = _sc_segsum(n_pad, d, k)(h, src, dst)

  y = pl.pallas_call(
      _tc_layer2,
      grid=grid,
      in_specs=[
          pl.BlockSpec((_NC, blk, d), lambda i: (0, i, 0)),
          pl.BlockSpec((_NC, blk, d), lambda i: (0, i, 0)),
          pl.BlockSpec((blk, d), lambda i: (i, 0)),
          pl.BlockSpec((d, d), lambda i: (0, 0)),
          pl.BlockSpec((d, d), lambda i: (0, 0)),
          pl.BlockSpec((1, d), lambda i: (0, 0)),
          pl.BlockSpec((d, 1), lambda i: (0, 0)),
          pl.BlockSpec((1, 1), lambda i: (0, 0)),
      ],
      out_specs=pl.BlockSpec((blk, 1), lambda i: (i, 0)),
      out_shape=jax.ShapeDtypeStruct((n_pad, 1), jnp.float32),
  )(agg2, cnt, h, W2l, W2r, b2.reshape(1, d), Wh, bh.reshape(1, 1))

  return y[:n, 0]


# speedup vs baseline: 1.0172x; 1.0172x over previous
import jax
import jax.numpy as jnp
from jax import lax
from jax.experimental import pallas as pl
from jax.experimental.pallas import tpu as pltpu
from jax.experimental.pallas import tpu_sc as plsc

_NC = 2
_NS = 16
_NW = _NC * _NS
_CHUNK = 32
_G = 8
_NBUF = 4


def _sc_segsum(n_pad, d, k):
  rps = n_pad // _NS

  def body(x_hbm, src_hbm, dst_hbm, agg_hbm,
           idx_s, idx_d, r0, r1, r2, r3, agg_sh, s0, s1, s2, s3):
    cid = lax.axis_index("c")
    sid = lax.axis_index("s")
    w = cid * _NS + sid
    row_bufs = (r0, r1, r2, r3)
    sems = (s0, s1, s2, s3)

    def zrow(i, _):
      for j in range(d // 16):
        r0[i, pl.ds(j * 16, 16)] = jnp.zeros((16,), jnp.float32)
      return 0
    lax.fori_loop(0, _CHUNK, zrow, 0)
    for t in range(rps // _CHUNK):
      pltpu.sync_copy(r0, agg_sh.at[pl.ds(sid * rps + t * _CHUNK, _CHUNK)])
    plsc.subcore_barrier()

    def stage(g):
      gp = g % 2
      pltpu.sync_copy(src_hbm.at[w, pl.ds(g * _G, _G)], idx_s.at[gp])
      pltpu.sync_copy(dst_hbm.at[w, pl.ds(g * _G, _G)], idx_d.at[gp])

    def start_gather(c, b):
      isrc = idx_s.at[(c // _G) % 2, c % _G]
      pltpu.async_copy(x_hbm.at[isrc], row_bufs[b], sems[b])

    def finish(c, b):
      isrc = idx_s.at[(c // _G) % 2, c % _G]
      pltpu.make_async_copy(x_hbm.at[isrc], row_bufs[b], sems[b]).wait()
      pltpu.sync_copy(row_bufs[b], agg_sh.at[idx_d.at[(c // _G) % 2, c % _G]],
                      add=True)

    stage(0)
    for b in range(_NBUF):
      start_gather(b, b)

    nq = k // _NBUF
    def quad(q, _):
      c0 = _NBUF * q
      @pl.when(jnp.logical_and(q % 2 == 1, q < nq - 1))
      def _():
        stage((c0 + _NBUF) // _G)
      for u in range(_NBUF):
        finish(c0 + u, u)
        @pl.when(q < nq - 1)
        def _():
          start_gather(c0 + u + _NBUF, u)
      return 0
    lax.fori_loop(0, nq, quad, 0)
    plsc.subcore_barrier()

    for t in range(rps // _CHUNK):
      sl = pl.ds(sid * rps + t * _CHUNK, _CHUNK)
      pltpu.sync_copy(agg_sh.at[sl], r0)
      pltpu.sync_copy(r0, agg_hbm.at[cid, sl])

  mesh = plsc.VectorSubcoreMesh(
      core_axis_name="c", subcore_axis_name="s",
      num_cores=_NC, num_subcores=_NS)
  return pl.kernel(
      body,
      out_type=jax.ShapeDtypeStruct((_NC, n_pad, d), jnp.float32),
      mesh=mesh,
      scratch_types=[
          pltpu.VMEM((2, _G, _CHUNK), jnp.int32),
          pltpu.VMEM((2, _G, _CHUNK), jnp.int32),
          pltpu.VMEM((_CHUNK, d), jnp.float32),
          pltpu.VMEM((_CHUNK, d), jnp.float32),
          pltpu.VMEM((_CHUNK, d), jnp.float32),
          pltpu.VMEM((_CHUNK, d), jnp.float32),
          pltpu.VMEM_SHARED((n_pad, d), jnp.float32),
          pltpu.SemaphoreType.DMA,
          pltpu.SemaphoreType.DMA,
          pltpu.SemaphoreType.DMA,
          pltpu.SemaphoreType.DMA,
      ],
  )


def _sc_count(n_pad, d, k, chunk):
  rps = n_pad // _NS
  ng = k // _G

  def body(dst_hbm, cnt_hbm, idx_d, ones, cnt_sh):
    cid = lax.axis_index("c")
    sid = lax.axis_index("s")
    w = cid * _NS + sid

    def zrow(i, _):
      for j in range(d // 16):
        ones[i, pl.ds(j * 16, 16)] = jnp.zeros((16,), jnp.float32)
      return 0
    lax.fori_loop(0, chunk, zrow, 0)
    for t in range(rps // chunk):
      pltpu.sync_copy(ones, cnt_sh.at[pl.ds(sid * rps + t * chunk, chunk)])

    def orow(i, _):
      for j in range(d // 16):
        ones[i, pl.ds(j * 16, 16)] = jnp.ones((16,), jnp.float32)
      return 0
    lax.fori_loop(0, chunk, orow, 0)
    plsc.subcore_barrier()

    def group(g, _):
      pltpu.sync_copy(dst_hbm.at[w, pl.ds(g * _G, _G)], idx_d)
      for j in range(_G):
        pltpu.sync_copy(ones, cnt_sh.at[idx_d.at[j]], add=True)
      return 0
    lax.fori_loop(0, ng, group, 0)
    plsc.subcore_barrier()

    for t in range(rps // chunk):
      sl = pl.ds(sid * rps + t * chunk, chunk)
      pltpu.sync_copy(cnt_sh.at[sl], ones)
      pltpu.sync_copy(ones, cnt_hbm.at[cid, sl])

  mesh = plsc.VectorSubcoreMesh(
      core_axis_name="c", subcore_axis_name="s",
      num_cores=_NC, num_subcores=_NS)
  return pl.kernel(
      body,
      out_type=jax.ShapeDtypeStruct((_NC, n_pad, d), jnp.float32),
      mesh=mesh,
      scratch_types=[
          pltpu.VMEM((_G, chunk), jnp.int32),
          pltpu.VMEM((chunk, d), jnp.float32),
          pltpu.VMEM_SHARED((n_pad, d), jnp.float32),
      ],
  )


def _tc_layer1(agg_ref, cnt_ref, x_ref, wl_ref, wr_ref, b_ref, o_ref):
  agg = agg_ref[0] + agg_ref[1]
  cnt = cnt_ref[0, :, 0:1] + cnt_ref[1, :, 0:1]
  mean = agg / jnp.maximum(cnt, 1.0)
  acc = jnp.dot(mean, wl_ref[...], preferred_element_type=jnp.float32)
  acc += jnp.dot(x_ref[...], wr_ref[...], preferred_element_type=jnp.float32)
  o_ref[...] = jnp.maximum(acc + b_ref[...], 0.0)


def _tc_layer2(agg_ref, cnt_ref, h_ref, wl_ref, wr_ref, b_ref, wh_ref, bh_ref,
               o_ref):
  agg = agg_ref[0] + agg_ref[1]
  cnt = cnt_ref[0, :, 0:1] + cnt_ref[1, :, 0:1]
  mean = agg / jnp.maximum(cnt, 1.0)
  acc = jnp.dot(mean, wl_ref[...], preferred_element_type=jnp.float32)
  acc += jnp.dot(h_ref[...], wr_ref[...], preferred_element_type=jnp.float32)
  h2 = jnp.maximum(acc + b_ref[...], 0.0)
  o_ref[...] = jnp.dot(h2, wh_ref[...], preferred_element_type=jnp.float32) \
      + bh_ref[...]


def kernel(x, edge_index, W1l, b1, W1r, W2l, b2, W2r, Wh, bh):
  n, d = x.shape
  e = edge_index.shape[1]

  k = -(-e // (_NW * _CHUNK * _G)) * _G
  e_pad = _NW * _CHUNK * k
  n_pad = -(-n // (_NS * _CHUNK)) * (_NS * _CHUNK)

  src = jnp.pad(edge_index[0], (0, e_pad - e)).reshape(_NW, k, _CHUNK)
  dst = jnp.pad(edge_index[1], (0, e_pad - e),
                constant_values=n).reshape(_NW, k, _CHUNK)
  x_pad = jnp.pad(x, ((0, n_pad - n), (0, 0)))

  cnt = _sc_count(n_pad, d, k // 2, 2 * _CHUNK)(
      dst.reshape(_NW, k // 2, 2 * _CHUNK))
  agg1 = _sc_segsum(n_pad, d, k)(x, src, dst)

  blk = 512
  grid = (n_pad // blk,)
  h = pl.pallas_call(
      _tc_layer1,
      grid=grid,
      in_specs=[
          pl.BlockSpec((_NC, blk, d), lambda i: (0, i, 0)),
          pl.BlockSpec((_NC, blk, d), lambda i: (0, i, 0)),
          pl.BlockSpec((blk, d), lambda i: (i, 0)),
          pl.BlockSpec((d, d), lambda i: (0, 0)),
          pl.BlockSpec((d, d), lambda i: (0, 0)),
          pl.BlockSpec((1, d), lambda i: (0, 0)),
      ],
      out_specs=pl.BlockSpec((blk, d), lambda i: (i, 0)),
      out_shape=jax.ShapeDtypeStruct((n_pad, d), jnp.float32),
  )(agg1, cnt, x_pad, W1l, W1r, b1.reshape(1, d))

  agg2 = _sc_segsum(n_pad, d, k)(h, src, dst)

  y = pl.pallas_call(
      _tc_layer2,
      grid=grid,
      in_specs=[
          pl.BlockSpec((_NC, blk, d), lambda i: (0, i, 0)),
          pl.BlockSpec((_NC, blk, d), lambda i: (0, i, 0)),
          pl.BlockSpec((blk, d), lambda i: (i, 0)),
          pl.BlockSpec((d, d), lambda i: (0, 0)),
          pl.BlockSpec((d, d), lambda i: (0, 0)),
          pl.BlockSpec((1, d), lambda i: (0, 0)),
          pl.BlockSpec((d, 1), lambda i: (0, 0)),
          pl.BlockSpec((1, 1), lambda i: (0, 0)),
      ],
      out_specs=pl.BlockSpec((blk, 1), lambda i: (i, 0)),
      out_shape=jax.ShapeDtypeStruct((n_pad, 1), jnp.float32),
  )(agg2, cnt, h, W2l, W2r, b2.reshape(1, d), Wh, bh.reshape(1, 1))

  return y[:n, 0]
